# baseline (speedup 1.0000x reference)
import jax
import jax.numpy as jnp
from jax import lax
from jax.experimental import pallas as pl
from jax.experimental.pallas import tpu as pltpu
from jax.experimental.pallas import tpu_sc as plsc

T = 4096
D = 2048
B = 4 * 4096
NC, NS = 2, 16
NW = NC * NS
BPW = B // NW
C = 8
NCH = BPW // C
NBUF = 4


def _gather_body(idx_hbm, table_hbm, out_hbm, idx_v, *rest):
    bufs, gsems, ssems = rest[:NBUF], rest[NBUF:2 * NBUF], rest[2 * NBUF:]
    wid = lax.axis_index("s") * NC + lax.axis_index("c")
    pltpu.sync_copy(idx_hbm.at[wid], idx_v)
    base = wid * BPW

    for b in range(NBUF):
        pltpu.async_copy(table_hbm.at[idx_v.at[b]], bufs[b], gsems[b])

    def step_group(k, carry):
        for b in range(NBUF):
            g = NBUF * k + b
            pltpu.make_async_copy(
                table_hbm.at[idx_v.at[g]], bufs[b], gsems[b]).wait()
            scat = pltpu.make_async_copy(
                bufs[b], out_hbm.at[pl.ds(base + g * C, C)], ssems[b])
            scat.start()
            scat.wait()

            @pl.when(g + NBUF < NCH)
            def _():
                pltpu.async_copy(
                    table_hbm.at[idx_v.at[g + NBUF]], bufs[b], gsems[b])

        return carry

    lax.fori_loop(0, NCH // NBUF, step_group, 0)


def kernel(pos, pos_embedding):
    idx = pos.reshape(NW, NCH, C).astype(jnp.int32)
    mesh = plsc.VectorSubcoreMesh(core_axis_name="c", subcore_axis_name="s")
    out = pl.kernel(
        _gather_body,
        mesh=mesh,
        out_type=jax.ShapeDtypeStruct((B, D), jnp.float32),
        scratch_types=(
            [pltpu.VMEM((NCH, C), jnp.int32)]
            + [pltpu.VMEM((C, D), jnp.float32)] * NBUF
            + [pltpu.SemaphoreType.DMA] * (2 * NBUF)
        ),
    )(idx, pos_embedding)
    return out.reshape(pos.shape[0], pos.shape[1], D)


# speedup vs baseline: 1.1335x; 1.1335x over previous
import jax
import jax.numpy as jnp
from jax import lax
from jax.experimental import pallas as pl
from jax.experimental.pallas import tpu as pltpu
from jax.experimental.pallas import tpu_sc as plsc

T = 4096
D = 2048
B = 4 * 4096
NC, NS = 2, 16
NW = NC * NS
RPT = T // NW
C = 16
NSL = RPT // C
NV = B // 16
CAP = B + 16


def _routed_body(idx_hbm, table_hbm, out_hbm,
                 idx_v, own_l, slice_l, buf0, buf1,
                 isem, gsem0, gsem1, ssem):
    wid = lax.axis_index("s") * NC + lax.axis_index("c")
    tbase = wid * RPT

    bufs = (buf0, buf1)
    gsems = (gsem0, gsem1)

    pltpu.async_copy(table_hbm.at[pl.ds(tbase, C)], buf0, gsem0)
    pltpu.async_copy(table_hbm.at[pl.ds(tbase + C, C)], buf1, gsem1)
    pltpu.async_copy(idx_hbm, idx_v, isem)
    pltpu.make_async_copy(idx_hbm, idx_v, isem).wait()

    lanes = lax.iota(jnp.int32, 16)

    def scan_step(i, off):
        x = idx_v[pl.ds(i * 16, 16)]
        rel = x - tbase
        m = (rel >= 0) & (rel < RPT)
        packed = lax.shift_left(i * 16 + lanes, 7) | rel
        plsc.store_compressed(own_l.at[pl.ds(off, 16)], packed, mask=m)
        cnt = plsc.all_reduce_population_count(m)[0]
        return off + cnt

    n_own = lax.fori_loop(0, NV, scan_step, 0)

    def do_slice(sl, buf, gsem):
        lo = sl * C

        def filt_step(k, off):
            v = own_l[pl.ds(k * 16, 16)]
            r = v & (RPT - 1)
            m = (r >= lo) & (r < lo + C) & (k * 16 + lanes < n_own)
            plsc.store_compressed(slice_l.at[pl.ds(off, 16)], v, mask=m)
            cnt = plsc.all_reduce_population_count(m)[0]
            return off + cnt

        n_sl = lax.fori_loop(0, (n_own + 15) // 16, filt_step, 0)

        pltpu.make_async_copy(
            table_hbm.at[pl.ds(tbase + lo, C)], buf, gsem).wait()

        def fire(k, carry):
            v16 = slice_l[pl.ds(k * 16, 16)]
            for j in range(16):
                @pl.when(k * 16 + j < n_sl)
                def _():
                    v = v16[j]
                    pos = lax.shift_right_logical(v, 7)
                    row = (v & (RPT - 1)) - lo
                    pltpu.make_async_copy(
                        buf.at[pl.ds(row, 1)], out_hbm.at[pl.ds(pos, 1)],
                        ssem).start()
            return carry

        lax.fori_loop(0, (n_sl + 15) // 16, fire, 0)

        def drain(j, carry):
            pltpu.make_async_copy(
                table_hbm.at[pl.ds(0, 1)], buf.at[pl.ds(0, 1)], ssem).wait()
            return carry

        lax.fori_loop(0, n_sl, drain, 0)

        @pl.when(sl + 2 < NSL)
        def _():
            pltpu.async_copy(
                table_hbm.at[pl.ds(tbase + (sl + 2) * C, C)], buf, gsem)

    def slice_pair(s2, carry):
        for b in range(2):
            do_slice(2 * s2 + b, bufs[b], gsems[b])
        return carry

    lax.fori_loop(0, NSL // 2, slice_pair, 0)


def kernel(pos, pos_embedding):
    idx = pos.reshape(B).astype(jnp.int32)
    mesh = plsc.VectorSubcoreMesh(core_axis_name="c", subcore_axis_name="s")
    out = pl.kernel(
        _routed_body,
        mesh=mesh,
        compiler_params=pltpu.CompilerParams(needs_layout_passes=False),
        out_type=jax.ShapeDtypeStruct((B, D), jnp.float32),
        scratch_types=[
            pltpu.VMEM((B,), jnp.int32),
            pltpu.VMEM((CAP,), jnp.int32),
            pltpu.VMEM((CAP,), jnp.int32),
            pltpu.VMEM((C, D), jnp.float32),
            pltpu.VMEM((C, D), jnp.float32),
            pltpu.SemaphoreType.DMA,
            pltpu.SemaphoreType.DMA,
            pltpu.SemaphoreType.DMA,
            pltpu.SemaphoreType.DMA,
        ],
    )(idx, pos_embedding)
    return out.reshape(pos.shape[0], pos.shape[1], D)


# speedup vs baseline: 1.1407x; 1.0063x over previous
import jax
import jax.numpy as jnp
from jax import lax
from jax.experimental import pallas as pl
from jax.experimental.pallas import tpu as pltpu
from jax.experimental.pallas import tpu_sc as plsc

T = 4096
D = 2048
R = 4
L = 4096
B = R * L
NC, NS = 2, 16
NW = NC * NS
RPT = T // NW
C = 16
NSL = RPT // C
CAP = B + 16


def _routed_body(idx_hbm, table_hbm, out_hbm,
                 idx_v, own_l, slice_l, buf0, buf1,
                 isem, gsem0, gsem1, ssem):
    wid = lax.axis_index("s") * NC + lax.axis_index("c")
    tbase = wid * RPT

    bufs = (buf0, buf1)
    gsems = (gsem0, gsem1)

    pltpu.async_copy(table_hbm.at[pl.ds(tbase, C)], buf0, gsem0)
    pltpu.async_copy(table_hbm.at[pl.ds(tbase + C, C)], buf1, gsem1)
    pltpu.async_copy(idx_hbm, idx_v, isem)
    pltpu.make_async_copy(idx_hbm, idx_v, isem).wait()

    lanes = lax.iota(jnp.int32, 16)
    row_consts = [
        lax.shift_left(r * L + lanes, 7) for r in range(R)
    ]

    def scan_step(i, off):
        ibits = lax.shift_left(i, 11)
        for r in range(R):
            x = idx_v[r, pl.ds(i * 16, 16)]
            rel = x - tbase
            m = (rel >= 0) & (rel < RPT)
            packed = (row_consts[r] | rel) + ibits
            plsc.store_compressed(own_l.at[pl.ds(off, 16)], packed, mask=m)
            off = off + plsc.all_reduce_population_count(m)[0]
        return off

    n_own = lax.fori_loop(0, L // 16, scan_step, 0)

    def do_slice(sl, buf, gsem):
        lo = sl * C

        def filt_step(k, off):
            v = own_l[pl.ds(k * 16, 16)]
            r = v & (RPT - 1)
            m = (r >= lo) & (r < lo + C) & (k * 16 + lanes < n_own)
            plsc.store_compressed(slice_l.at[pl.ds(off, 16)], v, mask=m)
            return off + plsc.all_reduce_population_count(m)[0]

        n_sl = lax.fori_loop(0, (n_own + 15) // 16, filt_step, 0)

        pltpu.make_async_copy(
            table_hbm.at[pl.ds(tbase + lo, C)], buf, gsem).wait()

        def fire(k, carry):
            v16 = slice_l[pl.ds(k * 16, 16)]
            for j in range(16):
                @pl.when(k * 16 + j < n_sl)
                def _():
                    v = v16[j]
                    pos = lax.shift_right_logical(v, 7)
                    row = (v & (RPT - 1)) - lo
                    pltpu.make_async_copy(
                        buf.at[pl.ds(row, 1)], out_hbm.at[pl.ds(pos, 1)],
                        ssem).start()
            return carry

        lax.fori_loop(0, (n_sl + 15) // 16, fire, 0)

        def drain16(j, carry):
            pltpu.make_async_copy(
                table_hbm.at[pl.ds(0, C)], buf, ssem).wait()
            return carry

        def drain1(j, carry):
            pltpu.make_async_copy(
                table_hbm.at[pl.ds(0, 1)], buf.at[pl.ds(0, 1)], ssem).wait()
            return carry

        lax.fori_loop(0, n_sl // 16, drain16, 0)
        lax.fori_loop(0, n_sl & 15, drain1, 0)

        @pl.when(sl + 2 < NSL)
        def _():
            pltpu.async_copy(
                table_hbm.at[pl.ds(tbase + (sl + 2) * C, C)], buf, gsem)

    def slice_pair(s2, carry):
        for b in range(2):
            do_slice(2 * s2 + b, bufs[b], gsems[b])
        return carry

    lax.fori_loop(0, NSL // 2, slice_pair, 0)


def kernel(pos, pos_embedding):
    mesh = plsc.VectorSubcoreMesh(core_axis_name="c", subcore_axis_name="s")
    out = pl.kernel(
        _routed_body,
        mesh=mesh,
        compiler_params=pltpu.CompilerParams(needs_layout_passes=False),
        out_type=jax.ShapeDtypeStruct((B, D), jnp.float32),
        scratch_types=[
            pltpu.VMEM((R, L), jnp.int32),
            pltpu.VMEM((CAP,), jnp.int32),
            pltpu.VMEM((CAP,), jnp.int32),
            pltpu.VMEM((C, D), jnp.float32),
            pltpu.VMEM((C, D), jnp.float32),
            pltpu.SemaphoreType.DMA,
            pltpu.SemaphoreType.DMA,
            pltpu.SemaphoreType.DMA,
            pltpu.SemaphoreType.DMA,
        ],
    )(pos, pos_embedding)
    return out.reshape(pos.shape[0], pos.shape[1], D)
